# baseline (speedup 1.0000x reference)
import functools

import jax
import jax.numpy as jnp
from jax import lax
from jax.experimental import pallas as pl
from jax.experimental.pallas import tpu as pltpu
from jax.experimental.pallas import tpu_sc as plsc

BATCH = 16384
HIST = 50
DIM = 64
TOTAL = BATCH * HIST

NUM_CORES = 2
NUM_SUBCORES = 16
NUM_WORKERS = NUM_CORES * NUM_SUBCORES
B_PER_W = BATCH // NUM_WORKERS
IDS_PER_W = B_PER_W * HIST
CHUNK = 128
QSPLIT = B_PER_W // CHUNK
NUM_UNITS = HIST * QSPLIT
NBUF = 4
SBUF = 2
AHEAD = NBUF - 1


def _make_kernel():
    mesh = plsc.VectorSubcoreMesh(
        core_axis_name="c", subcore_axis_name="s", num_cores=NUM_CORES
    )

    @functools.partial(
        pl.kernel,
        mesh=mesh,
        out_type=jax.ShapeDtypeStruct((HIST, DIM, BATCH), jnp.float32),
        scratch_types=(
            [pltpu.VMEM((IDS_PER_W,), jnp.int32)]
            + [pltpu.VMEM((CHUNK,), jnp.int32)] * NBUF
            + [pltpu.VMEM((CHUNK,), jnp.int32)] * NBUF
            + [pltpu.VMEM((CHUNK, 128), jnp.float32)] * NBUF
            + [pltpu.VMEM((DIM, CHUNK), jnp.float32)] * SBUF
            + [pltpu.SemaphoreType.DMA] * (NBUF + SBUF)
        ),
        compiler_params=pltpu.CompilerParams(
            use_tc_tiling_on_sc=True, needs_layout_passes=False
        ),
    )
    def lookup(ids_hbm, table_hbm, out_hbm, ids_v, *rest):
        pidx = rest[:NBUF]
        hoff = rest[NBUF : 2 * NBUF]
        gbuf = rest[2 * NBUF : 3 * NBUF]
        oblk = rest[3 * NBUF : 3 * NBUF + SBUF]
        gsem = rest[3 * NBUF + SBUF : 4 * NBUF + SBUF]
        ssem = rest[4 * NBUF + SBUF :]

        wid = lax.axis_index("s") * NUM_CORES + lax.axis_index("c")
        ids_base = wid * IDS_PER_W
        col_base = wid * B_PER_W
        pltpu.sync_copy(ids_hbm.at[pl.ds(ids_base, IDS_PER_W)], ids_v)

        lane = lax.iota(jnp.int32, 16)
        lane50 = lane * HIST

        def build_indices(u, b):
            h = u // QSPLIT
            q = u % QSPLIT
            for g in range(CHUNK // 16):
                pos = (q * CHUNK + g * 16) * HIST + h + lane50
                ids16 = plsc.load_gather(ids_v, [pos])
                pidx[b][pl.ds(g * 16, 16)] = jnp.right_shift(ids16, 1)
                hoff[b][pl.ds(g * 16, 16)] = (ids16 & 1) * DIM

        def start_gather(b):
            pltpu.async_copy(table_hbm.at[pidx[b]], gbuf[b], gsem[b])

        def wait_gather(b):
            pltpu.make_async_copy(table_hbm.at[pidx[b]], gbuf[b], gsem[b]).wait()

        def permute(b, s):
            rows = [lane + cg * 16 for cg in range(CHUNK // 16)]
            cols = [hoff[b][pl.ds(cg * 16, 16)] for cg in range(CHUNK // 16)]

            def dblk(i, carry):
                d0 = i * 32
                for dd in range(32):
                    d = d0 + dd
                    for cg in range(CHUNK // 16):
                        v = plsc.load_gather(gbuf[b], [rows[cg], cols[cg] + d])
                        oblk[s][d, pl.ds(cg * 16, 16)] = v
                return carry

            lax.fori_loop(0, DIM // 32, dblk, 0)

        def out_slice(u):
            h = u // QSPLIT
            q = u % QSPLIT
            return out_hbm.at[h, :, pl.ds(col_base + q * CHUNK, CHUNK)]

        def start_store(u, s):
            pltpu.async_copy(oblk[s], out_slice(u), ssem[s])

        def drain_store(s):
            pltpu.make_async_copy(out_slice(0), oblk[s], ssem[s]).wait()

        def slot(t, b, s):
            wait_gather(b)

            @pl.when(t + AHEAD < NUM_UNITS)
            def _():
                b2 = (b + AHEAD) % NBUF
                build_indices(t + AHEAD, b2)
                start_gather(b2)

            @pl.when(t >= SBUF)
            def _():
                drain_store(s)

            permute(b, s)
            start_store(t, s)

        for u in range(AHEAD):
            build_indices(u, u % NBUF)
            start_gather(u % NBUF)

        GROUP = 4
        NGROUPS = NUM_UNITS // GROUP

        def group(gi, carry):
            for k in range(GROUP):
                slot(gi * GROUP + k, k % NBUF, k % SBUF)
            return carry

        lax.fori_loop(0, NGROUPS, group, 0)

        for s in range(SBUF):
            drain_store(s)

    return lookup


_lookup = _make_kernel()


@jax.jit
def kernel(token_ids, embedding_weights):
    flat_ids = token_ids.reshape(TOTAL)
    table2 = embedding_weights.reshape(500000, 128)
    out = _lookup(flat_ids, table2)
    return out.transpose(2, 0, 1)


# speedup vs baseline: 1.4781x; 1.4781x over previous
import functools

import jax
import jax.numpy as jnp
from jax import lax
from jax.experimental import pallas as pl
from jax.experimental.pallas import tpu as pltpu
from jax.experimental.pallas import tpu_sc as plsc

BATCH = 16384
HIST = 50
DIM = 64
TOTAL = BATCH * HIST

NUM_CORES = 2
NUM_SUBCORES = 16
NUM_WORKERS = NUM_CORES * NUM_SUBCORES
ROWS_PER_WORKER = TOTAL // NUM_WORKERS
CHUNK = 128
NUM_CHUNKS = ROWS_PER_WORKER // CHUNK
NBUF = 8
LAG = 6


def _make_kernel():
    mesh = plsc.VectorSubcoreMesh(
        core_axis_name="c", subcore_axis_name="s", num_cores=NUM_CORES
    )

    @functools.partial(
        pl.kernel,
        mesh=mesh,
        out_type=jax.ShapeDtypeStruct((TOTAL, DIM), jnp.float32),
        scratch_types=(
            [pltpu.VMEM((ROWS_PER_WORKER,), jnp.int32)]
            + [pltpu.VMEM((CHUNK, DIM), jnp.float32)] * NBUF
            + [pltpu.SemaphoreType.DMA] * (2 * NBUF)
        ),
        compiler_params=pltpu.CompilerParams(use_tc_tiling_on_sc=False),
    )
    def lookup(ids_hbm, table_hbm, out_hbm, idx_v, *rest):
        bufs = rest[:NBUF]
        gsem = rest[NBUF : 2 * NBUF]
        ssem = rest[2 * NBUF :]

        wid = lax.axis_index("s") * NUM_CORES + lax.axis_index("c")
        base = wid * ROWS_PER_WORKER
        pltpu.sync_copy(ids_hbm.at[pl.ds(base, ROWS_PER_WORKER)], idx_v)

        def idx_slice(t):
            return idx_v.at[pl.ds(t * CHUNK, CHUNK)]

        def start_gather(t, b):
            pltpu.async_copy(table_hbm.at[idx_slice(t)], bufs[b], gsem[b])

        def wait_gather(t, b):
            pltpu.make_async_copy(
                table_hbm.at[idx_slice(t)], bufs[b], gsem[b]
            ).wait()

        def start_store(t, b):
            pltpu.async_copy(
                bufs[b], out_hbm.at[pl.ds(base + t * CHUNK, CHUNK)], ssem[b]
            )

        def drain_store(b):
            pltpu.make_async_copy(
                out_hbm.at[pl.ds(base, CHUNK)], bufs[b], ssem[b]
            ).wait()

        for t in range(LAG):
            start_gather(t, t)
        for t in range(LAG, NBUF):
            wait_gather(t - LAG, t - LAG)
            start_store(t - LAG, t - LAG)
            start_gather(t, t)

        def group(g, carry):
            for b in range(NBUF):
                t = g * NBUF + b
                drain_store(b)
                start_gather(t, b)
                b2 = (b + NBUF - LAG) % NBUF
                wait_gather(t - LAG, b2)
                start_store(t - LAG, b2)
            return carry

        lax.fori_loop(1, NUM_CHUNKS // NBUF, group, 0)

        for t in range(NUM_CHUNKS, NUM_CHUNKS + LAG):
            t2 = t - LAG
            wait_gather(t2, t2 % NBUF)
            start_store(t2, t2 % NBUF)
        for b in range(NBUF):
            drain_store(b)

    return lookup


_lookup = _make_kernel()


@jax.jit
def kernel(token_ids, embedding_weights):
    flat_ids = token_ids.reshape(TOTAL)
    out = _lookup(flat_ids, embedding_weights)
    return out.reshape(BATCH, HIST, DIM)
